# baseline (speedup 1.0000x reference)
import jax
import jax.numpy as jnp
from jax import lax
from jax.experimental import pallas as pl
from jax.experimental.pallas import tpu as pltpu
from jax.experimental.pallas import tpu_sc as plsc

BATCH = 16384
EMBED = 64
NC = 2
NS = 16
LANES = 16
NW = NC * NS
BPW = BATCH // NW
CHUNK = 64
NCHUNK = BPW // CHUNK
FLAT = BPW * EMBED


def _mf_body(uid_hbm, sid_hbm, utab_hbm, stab_hbm, out_hbm,
             uids_v, sids_v, sh_u, sh_s, ubuf, sbuf, outv,
             sem_u, sem_s, sem_cp):
    sid_ax = lax.axis_index("s")
    wid = sid_ax * NC + lax.axis_index("c")
    base = wid * BPW

    pltpu.sync_copy(uid_hbm.at[pl.ds(base, BPW)], uids_v)
    pltpu.sync_copy(sid_hbm.at[pl.ds(base, BPW)], sids_v)

    def fire_16(t, _):
        uvec = uids_v[pl.ds(t * LANES, LANES)]
        svec = sids_v[pl.ds(t * LANES, LANES)]
        for j in range(LANES):
            uid = uvec[j]
            sid = svec[j]
            k = t * LANES + j
            pltpu.async_copy(
                utab_hbm.at[uid],
                sh_u.at[sid_ax, pl.ds(k * EMBED, EMBED)], sem_u)
            pltpu.async_copy(
                stab_hbm.at[sid],
                sh_s.at[sid_ax, pl.ds(k * EMBED, EMBED)], sem_s)
        return _

    lax.fori_loop(0, BPW // LANES, fire_16, None)

    def drain_one(k, _):
        pltpu.make_async_copy(
            utab_hbm.at[0],
            sh_u.at[sid_ax, pl.ds(0, EMBED)], sem_u).wait()
        pltpu.make_async_copy(
            stab_hbm.at[0],
            sh_s.at[sid_ax, pl.ds(0, EMBED)], sem_s).wait()
        return _

    lax.fori_loop(0, BPW, drain_one, None)

    lane = lax.iota(jnp.int32, LANES)

    def chunk_step(ci, _):
        cw = CHUNK * EMBED
        cu = pltpu.async_copy(sh_u.at[sid_ax, pl.ds(ci * cw, cw)],
                              ubuf, sem_cp)
        cs = pltpu.async_copy(sh_s.at[sid_ax, pl.ds(ci * cw, cw)],
                              sbuf, sem_cp)
        cu.wait()
        cs.wait()

        def group(t, _):
            flat = (t * LANES + lane) * EMBED
            acc0 = jnp.zeros((LANES,), jnp.float32)
            acc1 = jnp.zeros((LANES,), jnp.float32)
            for j in range(0, EMBED, 2):
                u0 = plsc.load_gather(ubuf, [flat + j])
                s0 = plsc.load_gather(sbuf, [flat + j])
                u1 = plsc.load_gather(ubuf, [flat + (j + 1)])
                s1 = plsc.load_gather(sbuf, [flat + (j + 1)])
                acc0 = acc0 + u0 * s0
                acc1 = acc1 + u1 * s1
            dot = acc0 + acc1
            rating = 10.0 / (1.0 + jnp.exp(-dot))
            outv[pl.ds(ci * CHUNK + t * LANES, LANES)] = rating
            return _

        lax.fori_loop(0, CHUNK // LANES, group, None)
        return _

    lax.fori_loop(0, NCHUNK, chunk_step, None)

    pltpu.sync_copy(outv, out_hbm.at[pl.ds(base, BPW)])


def kernel(user_id, song_id, user_embedding, song_embedding):
    mesh = plsc.VectorSubcoreMesh(core_axis_name="c", subcore_axis_name="s")
    k = pl.kernel(
        _mf_body,
        mesh=mesh,
        compiler_params=pltpu.CompilerParams(
            needs_layout_passes=False, use_tc_tiling_on_sc=True),
        out_type=jax.ShapeDtypeStruct((BATCH,), jnp.float32),
        scratch_types=[
            pltpu.VMEM((BPW,), jnp.int32),
            pltpu.VMEM((BPW,), jnp.int32),
            pltpu.VMEM_SHARED((NS, FLAT), jnp.float32),
            pltpu.VMEM_SHARED((NS, FLAT), jnp.float32),
            pltpu.VMEM((CHUNK * EMBED,), jnp.float32),
            pltpu.VMEM((CHUNK * EMBED,), jnp.float32),
            pltpu.VMEM((BPW,), jnp.float32),
            pltpu.SemaphoreType.DMA,
            pltpu.SemaphoreType.DMA,
            pltpu.SemaphoreType.DMA,
        ],
    )
    return k(user_id.astype(jnp.int32), song_id.astype(jnp.int32),
             user_embedding, song_embedding)


# speedup vs baseline: 1.3858x; 1.3858x over previous
import jax
import jax.numpy as jnp
from jax import lax
from jax.experimental import pallas as pl
from jax.experimental.pallas import tpu as pltpu
from jax.experimental.pallas import tpu_sc as plsc

BATCH = 16384
EMBED = 64
NC = 2
NS = 16
LANES = 16
NW = NC * NS
BPW = BATCH // NW
CHUNK = 128
NCHUNK = BPW // CHUNK


def _mf_body(uid_hbm, sid_hbm, utab_hbm, stab_hbm, out_hbm,
             uids_v, sids_v,
             ubuf0, ubuf1, sbuf0, sbuf1, outv,
             sem_u0, sem_u1, sem_s0, sem_s1):
    wid = lax.axis_index("s") * NC + lax.axis_index("c")
    base = wid * BPW

    pltpu.sync_copy(uid_hbm.at[pl.ds(base, BPW)], uids_v)
    pltpu.sync_copy(sid_hbm.at[pl.ds(base, BPW)], sids_v)

    ubufs = (ubuf0, ubuf1)
    sbufs = (sbuf0, sbuf1)
    usems = (sem_u0, sem_u1)
    ssems = (sem_s0, sem_s1)

    def fire(ci):
        ub, sb = ubufs[ci % 2], sbufs[ci % 2]
        us, ss = usems[ci % 2], ssems[ci % 2]

        def fire_16(t, _):
            uvec = uids_v[pl.ds(ci * CHUNK + t * LANES, LANES)]
            svec = sids_v[pl.ds(ci * CHUNK + t * LANES, LANES)]
            for j in range(LANES):
                uid = uvec[j]
                sid = svec[j]
                pltpu.async_copy(utab_hbm.at[pl.ds(uid, 1), :],
                                 ub.at[pl.ds(t * LANES + j, 1), :], us)
                pltpu.async_copy(stab_hbm.at[pl.ds(sid, 1), :],
                                 sb.at[pl.ds(t * LANES + j, 1), :], ss)
            return _

        lax.fori_loop(0, CHUNK // LANES, fire_16, None)

    def drain(ci):
        ub, sb = ubufs[ci % 2], sbufs[ci % 2]
        us, ss = usems[ci % 2], ssems[ci % 2]

        def drain_one(k, _):
            pltpu.make_async_copy(utab_hbm.at[pl.ds(0, 1), :],
                                  ub.at[pl.ds(0, 1), :], us).wait()
            pltpu.make_async_copy(stab_hbm.at[pl.ds(0, 1), :],
                                  sb.at[pl.ds(0, 1), :], ss).wait()
            return _

        lax.fori_loop(0, CHUNK, drain_one, None)

    lane = lax.iota(jnp.int32, LANES)

    def compute(ci):
        ub, sb = ubufs[ci % 2], sbufs[ci % 2]

        def group(t, _):
            ps = []
            for r in range(LANES):
                urow = ub.at[t * LANES + r]
                srow = sb.at[t * LANES + r]
                p = None
                for c in range(EMBED // LANES):
                    uv = urow[pl.ds(c * LANES, LANES)]
                    sv = srow[pl.ds(c * LANES, LANES)]
                    pr = uv * sv
                    p = pr if p is None else p + pr
                ps.append(p)
            k = 1
            while len(ps) > 1:
                idx = jnp.bitwise_xor(lane, k)
                mask = jnp.bitwise_and(lane, k) == 0
                nxt = []
                for i in range(0, len(ps), 2):
                    a, b = ps[i], ps[i + 1]
                    pa = a.at[idx].get(mode="promise_in_bounds")
                    pb = b.at[idx].get(mode="promise_in_bounds")
                    nxt.append(jnp.where(mask, a + pa, b + pb))
                ps = nxt
                k *= 2
            dot = ps[0]
            rating = 10.0 / (1.0 + jnp.exp(-dot))
            outv[pl.ds(ci * CHUNK + t * LANES, LANES)] = rating
            return _

        lax.fori_loop(0, CHUNK // LANES, group, None)

    fire(0)
    for ci in range(NCHUNK):
        if ci + 1 < NCHUNK:
            fire(ci + 1)
        drain(ci)
        compute(ci)

    pltpu.sync_copy(outv, out_hbm.at[pl.ds(base, BPW)])


def kernel(user_id, song_id, user_embedding, song_embedding):
    mesh = plsc.VectorSubcoreMesh(core_axis_name="c", subcore_axis_name="s")
    k = pl.kernel(
        _mf_body,
        mesh=mesh,
        compiler_params=pltpu.CompilerParams(
            needs_layout_passes=False, use_tc_tiling_on_sc=True),
        out_type=jax.ShapeDtypeStruct((BATCH,), jnp.float32),
        scratch_types=[
            pltpu.VMEM((BPW,), jnp.int32),
            pltpu.VMEM((BPW,), jnp.int32),
            pltpu.VMEM((CHUNK, EMBED), jnp.float32),
            pltpu.VMEM((CHUNK, EMBED), jnp.float32),
            pltpu.VMEM((CHUNK, EMBED), jnp.float32),
            pltpu.VMEM((CHUNK, EMBED), jnp.float32),
            pltpu.VMEM((BPW,), jnp.float32),
            pltpu.SemaphoreType.DMA,
            pltpu.SemaphoreType.DMA,
            pltpu.SemaphoreType.DMA,
            pltpu.SemaphoreType.DMA,
        ],
    )
    return k(user_id.astype(jnp.int32), song_id.astype(jnp.int32),
             user_embedding, song_embedding)
